# baseline (speedup 1.0000x reference)
import jax
import jax.numpy as jnp
from jax.experimental import pallas as pl

_BLOCK_ROWS = 16384


def _body(x_ref, w_ref, b_ref, gamma_ref, beta_ref, mean_ref, var_ref, o_ref):
    eps = 1e-3
    scale = gamma_ref[:] * jax.lax.rsqrt(var_ref[:] + eps)
    shift = (b_ref[:] - mean_ref[:]) * scale + beta_ref[:]
    acc = jnp.dot(x_ref[:], w_ref[:], preferred_element_type=jnp.float32)
    o_ref[:] = acc * scale + shift


def kernel(encoded_trajectories, W, b, gamma, beta, moving_mean, moving_var):
    x = encoded_trajectories
    B, N, D = x.shape
    U = W.shape[1]
    rows = B * N
    x2 = x.reshape(rows, D)

    vec = lambda v: v.reshape(1, U)
    br = _BLOCK_ROWS
    grid = (rows // br,)

    out = pl.pallas_call(
        _body,
        grid=grid,
        in_specs=[
            pl.BlockSpec((br, D), lambda i: (i, 0)),
            pl.BlockSpec((D, U), lambda i: (0, 0)),
            pl.BlockSpec((1, U), lambda i: (0, 0)),
            pl.BlockSpec((1, U), lambda i: (0, 0)),
            pl.BlockSpec((1, U), lambda i: (0, 0)),
            pl.BlockSpec((1, U), lambda i: (0, 0)),
            pl.BlockSpec((1, U), lambda i: (0, 0)),
        ],
        out_specs=pl.BlockSpec((br, U), lambda i: (i, 0)),
        out_shape=jax.ShapeDtypeStruct((rows, U), x.dtype),
        input_output_aliases={0: 0},
    )(x2, W, vec(b), vec(gamma), vec(beta), vec(moving_mean), vec(moving_var))

    return out.reshape(B, N, U)


# speedup vs baseline: 1.9951x; 1.9951x over previous
import jax
import jax.numpy as jnp
from jax.experimental import pallas as pl
from jax.experimental.pallas import tpu as pltpu

_BLOCK_ROWS = 16384


def _body(x_ref, w_ref, b_ref, gamma_ref, beta_ref, mean_ref, var_ref, o_ref):
    eps = 1e-3
    scale = gamma_ref[:] * jax.lax.rsqrt(var_ref[:] + eps)
    shift = (b_ref[:] - mean_ref[:]) * scale + beta_ref[:]
    acc = jnp.dot(x_ref[:], w_ref[:], preferred_element_type=jnp.float32)
    o_ref[:] = acc * scale + shift


def kernel(encoded_trajectories, W, b, gamma, beta, moving_mean, moving_var):
    x = encoded_trajectories
    B, N, D = x.shape
    U = W.shape[1]
    rows = B * N
    x2 = x.reshape(rows, D)

    vec = lambda v: v.reshape(1, U)
    br = _BLOCK_ROWS
    grid = (rows // br,)

    out = pl.pallas_call(
        _body,
        grid=grid,
        in_specs=[
            pl.BlockSpec((br, D), lambda i: (i, 0)),
            pl.BlockSpec((D, U), lambda i: (0, 0)),
            pl.BlockSpec((1, U), lambda i: (0, 0)),
            pl.BlockSpec((1, U), lambda i: (0, 0)),
            pl.BlockSpec((1, U), lambda i: (0, 0)),
            pl.BlockSpec((1, U), lambda i: (0, 0)),
            pl.BlockSpec((1, U), lambda i: (0, 0)),
        ],
        out_specs=pl.BlockSpec((br, U), lambda i: (i, 0)),
        out_shape=jax.ShapeDtypeStruct((rows, U), x.dtype),
        compiler_params=pltpu.CompilerParams(
            dimension_semantics=("parallel",),
        ),
    )(x2, W, vec(b), vec(gamma), vec(beta), vec(moving_mean), vec(moving_var))

    return out.reshape(B, N, U)


# speedup vs baseline: 2.0037x; 1.0043x over previous
import jax
import jax.numpy as jnp
from jax.experimental import pallas as pl

_BLOCK_ROWS = 16384


def _body(x_ref, w_ref, b_ref, gamma_ref, beta_ref, mean_ref, var_ref, o_ref):
    eps = 1e-3
    scale = gamma_ref[:] * jax.lax.rsqrt(var_ref[:] + eps)
    shift = (b_ref[:] - mean_ref[:]) * scale + beta_ref[:]
    acc = jnp.dot(x_ref[:], w_ref[:], preferred_element_type=jnp.float32)
    o_ref[:] = acc * scale + shift


def kernel(encoded_trajectories, W, b, gamma, beta, moving_mean, moving_var):
    x = encoded_trajectories
    B, N, D = x.shape
    U = W.shape[1]
    rows = B * N
    x2 = x.reshape(rows, D)

    vec = lambda v: v.reshape(1, U)
    br = _BLOCK_ROWS
    grid = (rows // br,)

    out = pl.pallas_call(
        _body,
        grid=grid,
        in_specs=[
            pl.BlockSpec((br, D), lambda i: (i, 0)),
            pl.BlockSpec((D, U), lambda i: (0, 0)),
            pl.BlockSpec((1, U), lambda i: (0, 0)),
            pl.BlockSpec((1, U), lambda i: (0, 0)),
            pl.BlockSpec((1, U), lambda i: (0, 0)),
            pl.BlockSpec((1, U), lambda i: (0, 0)),
            pl.BlockSpec((1, U), lambda i: (0, 0)),
        ],
        out_specs=pl.BlockSpec((br, U), lambda i: (i, 0)),
        out_shape=jax.ShapeDtypeStruct((rows, U), x.dtype),
    )(x2, W, vec(b), vec(gamma), vec(beta), vec(moving_mean), vec(moving_var))

    return out.reshape(B, N, U)


# speedup vs baseline: 2.0224x; 1.0093x over previous
import jax
import jax.numpy as jnp
from jax.experimental import pallas as pl
from jax.experimental.pallas import tpu as pltpu

_CHUNK = 4096
_NBUF = 4


def _body(x_hbm, w_ref, b_ref, g_ref, be_ref, m_ref, v_ref, o_hbm,
          in_buf, out_buf, in_sem, out_sem):
    n_chunks = x_hbm.shape[0] // _CHUNK
    eps = 1e-3
    scale = g_ref[:] * jax.lax.rsqrt(v_ref[:] + eps)
    shift = (b_ref[:] - m_ref[:]) * scale + be_ref[:]
    w = w_ref[:]

    def in_copy(c, slot):
        return pltpu.make_async_copy(
            x_hbm.at[pl.ds(c * _CHUNK, _CHUNK), :],
            in_buf.at[slot], in_sem.at[slot])

    def out_copy(c, slot):
        return pltpu.make_async_copy(
            out_buf.at[slot],
            o_hbm.at[pl.ds(c * _CHUNK, _CHUNK), :], out_sem.at[slot])

    for s in range(_NBUF):
        in_copy(s, s).start()

    def step(c, carry):
        slot = jax.lax.rem(c, _NBUF)
        in_copy(c, slot).wait()

        @pl.when(c >= _NBUF)
        def _():
            out_copy(c - _NBUF, slot).wait()

        acc = jnp.dot(in_buf[slot], w, preferred_element_type=jnp.float32)
        out_buf[slot] = acc * scale + shift
        out_copy(c, slot).start()

        @pl.when(c + _NBUF < n_chunks)
        def _():
            in_copy(c + _NBUF, slot).start()

        return carry

    jax.lax.fori_loop(0, n_chunks, step, 0)

    for s in range(_NBUF):
        out_copy(n_chunks - _NBUF + s, s).wait()


def kernel(encoded_trajectories, W, b, gamma, beta, moving_mean, moving_var):
    x = encoded_trajectories
    B, N, D = x.shape
    U = W.shape[1]
    rows = B * N
    x2 = x.reshape(rows, D)

    vec = lambda v: v.reshape(1, U)
    any_spec = pl.BlockSpec(memory_space=pl.ANY)
    vmem = pl.BlockSpec(memory_space=pltpu.MemorySpace.VMEM)

    out = pl.pallas_call(
        _body,
        in_specs=[any_spec, vmem, vmem, vmem, vmem, vmem, vmem],
        out_specs=any_spec,
        out_shape=jax.ShapeDtypeStruct((rows, U), x.dtype),
        scratch_shapes=[
            pltpu.VMEM((_NBUF, _CHUNK, D), jnp.float32),
            pltpu.VMEM((_NBUF, _CHUNK, U), jnp.float32),
            pltpu.SemaphoreType.DMA((_NBUF,)),
            pltpu.SemaphoreType.DMA((_NBUF,)),
        ],
    )(x2, W, vec(b), vec(gamma), vec(beta), vec(moving_mean), vec(moving_var))

    return out.reshape(B, N, U)


# speedup vs baseline: 2.0235x; 1.0006x over previous
import jax
import jax.numpy as jnp
from jax.experimental import pallas as pl
from jax.experimental.pallas import tpu as pltpu

_CHUNK = 8192
_NBUF = 3


def _body(x_hbm, w_ref, b_ref, g_ref, be_ref, m_ref, v_ref, o_hbm,
          in_buf, out_buf, in_sem, out_sem):
    n_chunks = x_hbm.shape[0] // _CHUNK
    eps = 1e-3
    scale = g_ref[:] * jax.lax.rsqrt(v_ref[:] + eps)
    shift = (b_ref[:] - m_ref[:]) * scale + be_ref[:]
    w = w_ref[:]

    def in_copy(c, slot):
        return pltpu.make_async_copy(
            x_hbm.at[pl.ds(c * _CHUNK, _CHUNK), :],
            in_buf.at[slot], in_sem.at[slot])

    def out_copy(c, slot):
        return pltpu.make_async_copy(
            out_buf.at[slot],
            o_hbm.at[pl.ds(c * _CHUNK, _CHUNK), :], out_sem.at[slot])

    for s in range(_NBUF):
        in_copy(s, s).start()

    def step(c, carry):
        slot = jax.lax.rem(c, _NBUF)
        in_copy(c, slot).wait()

        @pl.when(c >= _NBUF)
        def _():
            out_copy(c - _NBUF, slot).wait()

        acc = jnp.dot(in_buf[slot], w, preferred_element_type=jnp.float32)
        out_buf[slot] = acc * scale + shift
        out_copy(c, slot).start()

        @pl.when(c + _NBUF < n_chunks)
        def _():
            in_copy(c + _NBUF, slot).start()

        return carry

    jax.lax.fori_loop(0, n_chunks, step, 0)

    for s in range(_NBUF):
        out_copy(n_chunks - _NBUF + s, s).wait()


def kernel(encoded_trajectories, W, b, gamma, beta, moving_mean, moving_var):
    x = encoded_trajectories
    B, N, D = x.shape
    U = W.shape[1]
    rows = B * N
    x2 = x.reshape(rows, D)

    vec = lambda v: v.reshape(1, U)
    any_spec = pl.BlockSpec(memory_space=pl.ANY)
    vmem = pl.BlockSpec(memory_space=pltpu.MemorySpace.VMEM)

    out = pl.pallas_call(
        _body,
        in_specs=[any_spec, vmem, vmem, vmem, vmem, vmem, vmem],
        out_specs=any_spec,
        out_shape=jax.ShapeDtypeStruct((rows, U), x.dtype),
        scratch_shapes=[
            pltpu.VMEM((_NBUF, _CHUNK, D), jnp.float32),
            pltpu.VMEM((_NBUF, _CHUNK, U), jnp.float32),
            pltpu.SemaphoreType.DMA((_NBUF,)),
            pltpu.SemaphoreType.DMA((_NBUF,)),
        ],
    )(x2, W, vec(b), vec(gamma), vec(beta), vec(moving_mean), vec(moving_var))

    return out.reshape(B, N, U)


# speedup vs baseline: 2.0293x; 1.0029x over previous
import jax
import jax.numpy as jnp
from jax.experimental import pallas as pl
from jax.experimental.pallas import tpu as pltpu

_CHUNK = 8192
_NBUF = 4


def _body(x_hbm, w_ref, b_ref, g_ref, be_ref, m_ref, v_ref, o_hbm,
          in_buf, out_buf, in_sem, out_sem):
    n_chunks = x_hbm.shape[0] // _CHUNK
    eps = 1e-3
    scale = g_ref[:] * jax.lax.rsqrt(v_ref[:] + eps)
    shift = (b_ref[:] - m_ref[:]) * scale + be_ref[:]
    w = w_ref[:]

    def in_copy(c, slot):
        return pltpu.make_async_copy(
            x_hbm.at[pl.ds(c * _CHUNK, _CHUNK), :],
            in_buf.at[slot], in_sem.at[slot])

    def out_copy(c, slot):
        return pltpu.make_async_copy(
            out_buf.at[slot],
            o_hbm.at[pl.ds(c * _CHUNK, _CHUNK), :], out_sem.at[slot])

    for s in range(_NBUF):
        in_copy(s, s).start()

    def step(c, carry):
        slot = jax.lax.rem(c, _NBUF)
        in_copy(c, slot).wait()

        @pl.when(c >= _NBUF)
        def _():
            out_copy(c - _NBUF, slot).wait()

        acc = jnp.dot(in_buf[slot], w, preferred_element_type=jnp.float32)
        out_buf[slot] = acc * scale + shift
        out_copy(c, slot).start()

        @pl.when(c + _NBUF < n_chunks)
        def _():
            in_copy(c + _NBUF, slot).start()

        return carry

    jax.lax.fori_loop(0, n_chunks, step, 0)

    for s in range(_NBUF):
        out_copy(n_chunks - _NBUF + s, s).wait()


def kernel(encoded_trajectories, W, b, gamma, beta, moving_mean, moving_var):
    x = encoded_trajectories
    B, N, D = x.shape
    U = W.shape[1]
    rows = B * N
    x2 = x.reshape(rows, D)

    vec = lambda v: v.reshape(1, U)
    any_spec = pl.BlockSpec(memory_space=pl.ANY)
    vmem = pl.BlockSpec(memory_space=pltpu.MemorySpace.VMEM)

    out = pl.pallas_call(
        _body,
        in_specs=[any_spec, vmem, vmem, vmem, vmem, vmem, vmem],
        out_specs=any_spec,
        out_shape=jax.ShapeDtypeStruct((rows, U), x.dtype),
        scratch_shapes=[
            pltpu.VMEM((_NBUF, _CHUNK, D), jnp.float32),
            pltpu.VMEM((_NBUF, _CHUNK, U), jnp.float32),
            pltpu.SemaphoreType.DMA((_NBUF,)),
            pltpu.SemaphoreType.DMA((_NBUF,)),
        ],
    )(x2, W, vec(b), vec(gamma), vec(beta), vec(moving_mean), vec(moving_var))

    return out.reshape(B, N, U)
